# Initial kernel scaffold; baseline (speedup 1.0000x reference)
import jax
import jax.numpy as jnp
from jax.experimental import pallas as pl
from jax.experimental.pallas import tpu as pltpu

BS = 8
NC = 16
C = 128
D = 64
K = 16
K_B = 8
B_BORDER = 16
H_MOD = 32
N = NC * C
MOD_IN = K + 3 * D + 1
MOD_OUT = K + K_B + 1 + D


def _cell_kernel(heb_ref, h_ref, dec_ref, prim_ref, w1_ref, b1_ref, w2_ref,
                 b2_ref, nid_ref, idx_ref, out_ref, sel_ref):
    heb = heb_ref[:, 0]
    hh = h_ref[:, 0]
    dec = dec_ref[:, 0]
    prim = prim_ref[:, 0]
    nid = nid_ref[0]
    nid_b = jnp.broadcast_to(nid[None], (BS, C, D))
    x = jnp.concatenate([heb, hh, dec[..., None], prim, nid_b], axis=-1)

    w1 = w1_ref[0]
    b1 = b1_ref[0]
    hid = jax.lax.dot_general(
        x, w1,
        dimension_numbers=(((2,), (2,)), ((1,), (0,))),
        preferred_element_type=jnp.float32)
    hid = jnp.tanh(hid + b1[:, None, :])

    w2 = w2_ref[0]
    b2 = b2_ref[0]
    out = jax.lax.dot_general(
        hid, w2,
        dimension_numbers=(((2,), (1,)), ((0,), (0,))),
        preferred_element_type=jnp.float32)
    out = out + b2[:, None, :]
    out_ref[0] = out

    idx = idx_ref[0, 0]
    iota = jax.lax.broadcasted_iota(jnp.int32, (B_BORDER, C), 1)
    onehot = (idx[:, None] == iota).astype(jnp.float32)
    border = out[:, :, K:K + K_B].reshape(C, BS * K_B)
    sel = jax.lax.dot_general(
        onehot, border,
        dimension_numbers=(((1,), (0,)), ((), ())),
        preferred_element_type=jnp.float32)
    sel_ref[0] = sel


def kernel(h, hebbian_traces, decay_logit, primitives, mod_w1, mod_b1,
           mod_w2, mod_b2, neuron_id, border_indices):
    w1 = mod_w1.reshape(NC, C, H_MOD, MOD_IN)
    b1 = mod_b1.reshape(NC, C, H_MOD)
    w2 = mod_w2.reshape(NC, C, H_MOD, MOD_OUT)
    b2 = mod_b2.reshape(NC, C, MOD_OUT)
    bidx = border_indices.reshape(NC, 1, B_BORDER).astype(jnp.int32)

    out, sel = pl.pallas_call(
        _cell_kernel,
        grid=(NC,),
        in_specs=[
            pl.BlockSpec((BS, 1, C, K), lambda c: (0, c, 0, 0)),
            pl.BlockSpec((BS, 1, C, D), lambda c: (0, c, 0, 0)),
            pl.BlockSpec((BS, 1, C), lambda c: (0, c, 0)),
            pl.BlockSpec((BS, 1, C, D), lambda c: (0, c, 0, 0)),
            pl.BlockSpec((1, C, H_MOD, MOD_IN), lambda c: (c, 0, 0, 0)),
            pl.BlockSpec((1, C, H_MOD), lambda c: (c, 0, 0)),
            pl.BlockSpec((1, C, H_MOD, MOD_OUT), lambda c: (c, 0, 0, 0)),
            pl.BlockSpec((1, C, MOD_OUT), lambda c: (c, 0, 0)),
            pl.BlockSpec((1, C, D), lambda c: (c, 0, 0)),
            pl.BlockSpec((1, 1, B_BORDER), lambda c: (c, 0, 0)),
        ],
        out_specs=[
            pl.BlockSpec((1, C, BS, MOD_OUT), lambda c: (c, 0, 0, 0)),
            pl.BlockSpec((1, B_BORDER, BS * K_B), lambda c: (c, 0, 0)),
        ],
        out_shape=[
            jax.ShapeDtypeStruct((NC, C, BS, MOD_OUT), jnp.float32),
            jax.ShapeDtypeStruct((NC, B_BORDER, BS * K_B), jnp.float32),
        ],
        compiler_params=pltpu.CompilerParams(
            dimension_semantics=("arbitrary",),
        ),
    )(hebbian_traces, h, decay_logit, primitives, w1, b1, w2, b2,
      neuron_id, bidx)

    out_t = jnp.transpose(out, (2, 0, 1, 3))
    new_w_conn = out_t[..., :K]
    new_decay = out_t[..., K + K_B]
    new_prim = out_t[..., K + K_B + 1:]
    sel_t = jnp.transpose(sel.reshape(NC, B_BORDER, BS, K_B), (2, 0, 1, 3))
    return (new_w_conn, sel_t, new_decay, new_prim)


# speedup vs baseline: 1.3182x; 1.3182x over previous
import jax
import jax.numpy as jnp
from jax.experimental import pallas as pl
from jax.experimental.pallas import tpu as pltpu

BS = 8
NC = 16
C = 128
D = 64
K = 16
K_B = 8
B_BORDER = 16
H_MOD = 32
N = NC * C
MOD_IN = K + 3 * D + 1
MOD_OUT = K + K_B + 1 + D


def _cell_kernel(heb_ref, h_ref, dec_ref, prim_ref, w1_ref, b1_ref, w2_ref,
                 b2_ref, nid_ref, idx_ref, out_ref, sel_ref):
    heb = heb_ref[:, 0]
    hh = h_ref[:, 0]
    dec = dec_ref[:, 0, 0]
    prim = prim_ref[:, 0]
    nid = nid_ref[0]
    nid_b = jnp.broadcast_to(nid[None], (BS, C, D))
    x = jnp.concatenate([heb, hh, dec[..., None], prim, nid_b], axis=-1)

    w1 = w1_ref[0]
    b1 = b1_ref[0]
    hid = jax.lax.dot_general(
        x, w1,
        dimension_numbers=(((2,), (2,)), ((1,), (0,))),
        preferred_element_type=jnp.float32)
    hid = jnp.tanh(hid + b1[:, None, :])

    w2 = w2_ref[0]
    b2 = b2_ref[0]
    out = jax.lax.dot_general(
        hid, w2,
        dimension_numbers=(((2,), (1,)), ((0,), (0,))),
        preferred_element_type=jnp.float32)
    out = out + b2[:, None, :]
    out_ref[0] = out

    idx = idx_ref[0, 0]
    iota = jax.lax.broadcasted_iota(jnp.int32, (B_BORDER, C), 1)
    onehot = (idx[:, None] == iota).astype(jnp.float32)
    border = out[:, :, K:K + K_B].reshape(C, BS * K_B)
    sel = jax.lax.dot_general(
        onehot, border,
        dimension_numbers=(((1,), (0,)), ((), ())),
        preferred_element_type=jnp.float32)
    sel_ref[0] = sel


def kernel(h, hebbian_traces, decay_logit, primitives, mod_w1, mod_b1,
           mod_w2, mod_b2, neuron_id, border_indices):
    w1 = mod_w1.reshape(NC, C, H_MOD, MOD_IN)
    b1 = mod_b1.reshape(NC, C, H_MOD)
    w2 = mod_w2.reshape(NC, C, H_MOD, MOD_OUT)
    b2 = mod_b2.reshape(NC, C, MOD_OUT)
    bidx = border_indices.reshape(NC, 1, B_BORDER).astype(jnp.int32)

    out, sel = pl.pallas_call(
        _cell_kernel,
        grid=(NC,),
        in_specs=[
            pl.BlockSpec((BS, 1, C, K), lambda c: (0, c, 0, 0)),
            pl.BlockSpec((BS, 1, C, D), lambda c: (0, c, 0, 0)),
            pl.BlockSpec((BS, 1, 1, C), lambda c: (0, c, 0, 0)),
            pl.BlockSpec((BS, 1, C, D), lambda c: (0, c, 0, 0)),
            pl.BlockSpec((1, C, H_MOD, MOD_IN), lambda c: (c, 0, 0, 0)),
            pl.BlockSpec((1, C, H_MOD), lambda c: (c, 0, 0)),
            pl.BlockSpec((1, C, H_MOD, MOD_OUT), lambda c: (c, 0, 0, 0)),
            pl.BlockSpec((1, C, MOD_OUT), lambda c: (c, 0, 0)),
            pl.BlockSpec((1, C, D), lambda c: (c, 0, 0)),
            pl.BlockSpec((1, 1, B_BORDER), lambda c: (c, 0, 0)),
        ],
        out_specs=[
            pl.BlockSpec((1, C, BS, MOD_OUT), lambda c: (c, 0, 0, 0)),
            pl.BlockSpec((1, B_BORDER, BS * K_B), lambda c: (c, 0, 0)),
        ],
        out_shape=[
            jax.ShapeDtypeStruct((NC, C, BS, MOD_OUT), jnp.float32),
            jax.ShapeDtypeStruct((NC, B_BORDER, BS * K_B), jnp.float32),
        ],
        compiler_params=pltpu.CompilerParams(
            dimension_semantics=("arbitrary",),
        ),
    )(hebbian_traces, h, decay_logit.reshape(BS, NC, 1, C), primitives,
      w1, b1, w2, b2, neuron_id, bidx)

    out_t = jnp.transpose(out, (2, 0, 1, 3))
    new_w_conn = out_t[..., :K]
    new_decay = out_t[..., K + K_B]
    new_prim = out_t[..., K + K_B + 1:]
    sel_t = jnp.transpose(sel.reshape(NC, B_BORDER, BS, K_B), (2, 0, 1, 3))
    return (new_w_conn, sel_t, new_decay, new_prim)


# speedup vs baseline: 1.4497x; 1.0998x over previous
import jax
import jax.numpy as jnp
from jax.experimental import pallas as pl
from jax.experimental.pallas import tpu as pltpu

BS = 8
NC = 16
C = 128
D = 64
K = 16
K_B = 8
B_BORDER = 16
H_MOD = 32
N = NC * C
MOD_IN = K + 3 * D + 1
MOD_OUT = K + K_B + 1 + D


def _cell_kernel(heb_ref, h_ref, dec_ref, prim_ref, w1_ref, b1_ref, w2_ref,
                 b2_ref, nid_ref, idx_ref,
                 wconn_ref, sel_ref, ndec_ref, nprim_ref):
    heb = heb_ref[:, 0]
    hh = h_ref[:, 0]
    dec = dec_ref[:, 0, 0]
    prim = prim_ref[:, 0]
    nid = nid_ref[0]
    nid_b = jnp.broadcast_to(nid[None], (BS, C, D))
    x = jnp.concatenate([heb, hh, dec[..., None], prim, nid_b], axis=-1)

    w1 = w1_ref[0]
    b1 = b1_ref[0]
    hid = jax.lax.dot_general(
        x, w1,
        dimension_numbers=(((2,), (2,)), ((1,), (0,))),
        preferred_element_type=jnp.float32)
    hid = jnp.tanh(hid + b1[:, None, :])

    w2 = w2_ref[0]
    b2 = b2_ref[0]
    out = jax.lax.dot_general(
        hid, w2,
        dimension_numbers=(((2,), (1,)), ((0,), (0,))),
        preferred_element_type=jnp.float32)
    out = out + b2[:, None, :]
    out_t = jnp.transpose(out, (1, 0, 2))
    wconn_ref[:, 0] = out_t[:, :, :K]
    ndec_ref[:, 0, 0] = out_t[:, :, K + K_B]
    nprim_ref[:, 0] = out_t[:, :, K + K_B + 1:]

    idx = idx_ref[0, 0]
    iota = jax.lax.broadcasted_iota(jnp.int32, (B_BORDER, C), 1)
    onehot = (idx[:, None] == iota).astype(jnp.float32)
    border = out_t[:, :, K:K + K_B]
    sel_bkj = jax.lax.dot_general(
        border, onehot,
        dimension_numbers=(((1,), (1,)), ((), ())),
        preferred_element_type=jnp.float32)
    sel_ref[:, 0] = jnp.transpose(sel_bkj, (0, 2, 1))


def kernel(h, hebbian_traces, decay_logit, primitives, mod_w1, mod_b1,
           mod_w2, mod_b2, neuron_id, border_indices):
    w1 = mod_w1.reshape(NC, C, H_MOD, MOD_IN)
    b1 = mod_b1.reshape(NC, C, H_MOD)
    w2 = mod_w2.reshape(NC, C, H_MOD, MOD_OUT)
    b2 = mod_b2.reshape(NC, C, MOD_OUT)
    bidx = border_indices.reshape(NC, 1, B_BORDER).astype(jnp.int32)

    wconn, sel, ndec, nprim = pl.pallas_call(
        _cell_kernel,
        grid=(NC,),
        in_specs=[
            pl.BlockSpec((BS, 1, C, K), lambda c: (0, c, 0, 0)),
            pl.BlockSpec((BS, 1, C, D), lambda c: (0, c, 0, 0)),
            pl.BlockSpec((BS, 1, 1, C), lambda c: (0, c, 0, 0)),
            pl.BlockSpec((BS, 1, C, D), lambda c: (0, c, 0, 0)),
            pl.BlockSpec((1, C, H_MOD, MOD_IN), lambda c: (c, 0, 0, 0)),
            pl.BlockSpec((1, C, H_MOD), lambda c: (c, 0, 0)),
            pl.BlockSpec((1, C, H_MOD, MOD_OUT), lambda c: (c, 0, 0, 0)),
            pl.BlockSpec((1, C, MOD_OUT), lambda c: (c, 0, 0)),
            pl.BlockSpec((1, C, D), lambda c: (c, 0, 0)),
            pl.BlockSpec((1, 1, B_BORDER), lambda c: (c, 0, 0)),
        ],
        out_specs=[
            pl.BlockSpec((BS, 1, C, K), lambda c: (0, c, 0, 0)),
            pl.BlockSpec((BS, 1, B_BORDER, K_B), lambda c: (0, c, 0, 0)),
            pl.BlockSpec((BS, 1, 1, C), lambda c: (0, c, 0, 0)),
            pl.BlockSpec((BS, 1, C, D), lambda c: (0, c, 0, 0)),
        ],
        out_shape=[
            jax.ShapeDtypeStruct((BS, NC, C, K), jnp.float32),
            jax.ShapeDtypeStruct((BS, NC, B_BORDER, K_B), jnp.float32),
            jax.ShapeDtypeStruct((BS, NC, 1, C), jnp.float32),
            jax.ShapeDtypeStruct((BS, NC, C, D), jnp.float32),
        ],
        compiler_params=pltpu.CompilerParams(
            dimension_semantics=("arbitrary",),
        ),
    )(hebbian_traces, h, decay_logit.reshape(BS, NC, 1, C), primitives,
      w1, b1, w2, b2, neuron_id, bidx)

    return (wconn, sel, ndec.reshape(BS, NC, C), nprim)
